# baseline (speedup 1.0000x reference)
import jax
import jax.numpy as jnp
from jax.experimental import pallas as pl
from jax.experimental.pallas import tpu as pltpu

BT = 2048


def _add_kernel(x_ref, pos_ref, out_ref):
    out_ref[0, :, :] = x_ref[0, :, :] + pos_ref[:, :]


def kernel(x, pos_table):
    B, T, C = x.shape
    grid = (T // BT, B)
    return pl.pallas_call(
        _add_kernel,
        grid=grid,
        in_specs=[
            pl.BlockSpec((1, BT, C), lambda t, b: (b, t, 0)),
            pl.BlockSpec((BT, C), lambda t, b: (t, 0)),
        ],
        out_specs=pl.BlockSpec((1, BT, C), lambda t, b: (b, t, 0)),
        out_shape=jax.ShapeDtypeStruct((B, T, C), x.dtype),
        compiler_params=pltpu.CompilerParams(
            dimension_semantics=("parallel", "arbitrary"),
        ),
    )(x, pos_table)
